# baseline (speedup 1.0000x reference)
import functools

import jax
import jax.numpy as jnp
from jax import lax
from jax.experimental import pallas as pl
from jax.experimental.pallas import tpu as pltpu
from jax.experimental.pallas import tpu_sc as plsc

N_NODES = 10000
IN_DIM = 128
HID_DIM = 128
OUT_DIM = 64
N_EDGES = 320000

NC = 2
NS = 16
NW = NC * NS
CHUNK = 128
CHUNKS_PER_TILE = 80
EDGES_PER_TILE = CHUNK * CHUNKS_PER_TILE
E_PAD = NW * EDGES_PER_TILE
N_PAD = 10240
ROWS_PER_TILE = N_PAD // NS
DUMMY_ROW = N_NODES

_mesh = plsc.VectorSubcoreMesh(core_axis_name="c", subcore_axis_name="s")
_sc_params = pltpu.CompilerParams(use_tc_tiling_on_sc=False)


def _deg_body(dst_hbm, deg_hbm, dstv, ones_v, stage_v, sem, acc):
    c = lax.axis_index("c")
    s = lax.axis_index("s")
    wid = c * NS + s
    pltpu.sync_copy(dst_hbm.at[wid], dstv)

    zero16 = jnp.zeros((16,), jnp.float32)
    one16 = jnp.ones((16,), jnp.float32)

    def fill_ones(r, _):
        ones_v[r, :] = one16
        return 0

    lax.fori_loop(0, CHUNK, fill_ones, 0)

    def fill_zero(r, _):
        stage_v[r, :] = zero16
        return 0

    lax.fori_loop(0, ROWS_PER_TILE, fill_zero, 0)
    pltpu.sync_copy(stage_v, acc.at[pl.ds(s * ROWS_PER_TILE, ROWS_PER_TILE)])
    plsc.subcore_barrier()

    def step(j, _):
        pltpu.async_copy(ones_v, acc.at[dstv.at[j]], sem, add=True)
        return 0

    lax.fori_loop(0, CHUNKS_PER_TILE, step, 0)

    def drain(j, _):
        pltpu.make_async_copy(ones_v, acc.at[dstv.at[0]], sem).wait()
        return 0

    lax.fori_loop(0, CHUNKS_PER_TILE, drain, 0)
    plsc.subcore_barrier()

    off = s * ROWS_PER_TILE
    pltpu.sync_copy(acc.at[pl.ds(off, ROWS_PER_TILE)],
                    deg_hbm.at[pl.ds(c * N_PAD + off, ROWS_PER_TILE)])


_deg_call = pl.kernel(
    _deg_body,
    out_type=jax.ShapeDtypeStruct((NC * N_PAD, 16), jnp.float32),
    mesh=_mesh,
    scratch_types=[
        pltpu.VMEM((CHUNKS_PER_TILE, CHUNK), jnp.int32),
        pltpu.VMEM((CHUNK, 16), jnp.float32),
        pltpu.VMEM((ROWS_PER_TILE, 16), jnp.float32),
        pltpu.SemaphoreType.DMA,
        pltpu.VMEM_SHARED((N_PAD, 16), jnp.float32),
    ],
    compiler_params=_sc_params,
)


AGG_D = 64


def _agg_pass(table_hbm, out_hbm, c, s, srcv, dstv, rows0, rows1, sem0, sem1,
              tbl, acc):
    ch = CHUNK
    npc = EDGES_PER_TILE // ch
    zero16 = jnp.zeros((16,), jnp.float32)

    def fill_zero(r, _):
        for k in range(AGG_D // 16):
            rows0[r, pl.ds(k * 16, 16)] = zero16
        return 0

    lax.fori_loop(0, ch, fill_zero, 0)
    base = s * ROWS_PER_TILE
    for k in range(ROWS_PER_TILE // ch):
        pltpu.sync_copy(rows0, acc.at[pl.ds(base + k * ch, ch)])
    pltpu.sync_copy(table_hbm.at[pl.ds(base, ROWS_PER_TILE)],
                    tbl.at[pl.ds(base, ROWS_PER_TILE)])
    plsc.subcore_barrier()

    pltpu.async_copy(tbl.at[srcv.at[0]], rows0, sem0)
    n2 = npc // 2

    def step(i, _):
        j0 = 2 * i
        pltpu.async_copy(tbl.at[srcv.at[j0 + 1]], rows1, sem1)
        pltpu.make_async_copy(tbl.at[srcv.at[j0]], rows0, sem0).wait()
        pltpu.sync_copy(rows0, acc.at[dstv.at[j0]], add=True)
        pltpu.async_copy(tbl.at[srcv.at[j0 + 2]], rows0, sem0)
        pltpu.make_async_copy(tbl.at[srcv.at[j0 + 1]], rows1, sem1).wait()
        pltpu.sync_copy(rows1, acc.at[dstv.at[j0 + 1]], add=True)
        return 0

    lax.fori_loop(0, n2, step, 0)
    pltpu.make_async_copy(tbl.at[srcv.at[npc]], rows0, sem0).wait()
    plsc.subcore_barrier()

    pltpu.sync_copy(acc.at[pl.ds(base, ROWS_PER_TILE)],
                    out_hbm.at[pl.ds(c * N_PAD + base, ROWS_PER_TILE)])


def _load_indices(src_hbm, dst_hbm, wid, srcv, dstv):
    npc = EDGES_PER_TILE // CHUNK
    pltpu.sync_copy(src_hbm.at[wid], srcv.at[pl.ds(0, npc)])
    pltpu.sync_copy(dst_hbm.at[wid], dstv)
    zero16i = jnp.zeros((16,), jnp.int32)
    for k in range(CHUNK // 16):
        srcv[npc, pl.ds(k * 16, 16)] = zero16i


_AGG_SCRATCH = [
    pltpu.VMEM((EDGES_PER_TILE // CHUNK + 1, CHUNK), jnp.int32),
    pltpu.VMEM((EDGES_PER_TILE // CHUNK, CHUNK), jnp.int32),
    pltpu.VMEM((CHUNK, AGG_D), jnp.float32),
    pltpu.VMEM((CHUNK, AGG_D), jnp.float32),
    pltpu.SemaphoreType.DMA,
    pltpu.SemaphoreType.DMA,
    pltpu.VMEM_SHARED((N_PAD, AGG_D), jnp.float32),
    pltpu.VMEM_SHARED((N_PAD, AGG_D), jnp.float32),
]


def _agg_body(table_hbm, src_hbm, dst_hbm, out_hbm, srcv, dstv, rows0, rows1,
              sem0, sem1, tbl, acc):
    c = lax.axis_index("c")
    s = lax.axis_index("s")
    _load_indices(src_hbm, dst_hbm, c * NS + s, srcv, dstv)
    _agg_pass(table_hbm, out_hbm, c, s, srcv, dstv, rows0, rows1, sem0, sem1,
              tbl, acc)


_agg64 = pl.kernel(
    _agg_body,
    out_type=jax.ShapeDtypeStruct((NC * N_PAD, AGG_D), jnp.float32),
    mesh=_mesh,
    scratch_types=list(_AGG_SCRATCH),
    compiler_params=_sc_params,
)


def _agg2_body(ta_hbm, tb_hbm, src_hbm, dst_hbm, outa_hbm, outb_hbm,
               srcv, dstv, rows0, rows1, sem0, sem1, tbl, acc):
    c = lax.axis_index("c")
    s = lax.axis_index("s")
    _load_indices(src_hbm, dst_hbm, c * NS + s, srcv, dstv)
    _agg_pass(ta_hbm, outa_hbm, c, s, srcv, dstv, rows0, rows1, sem0, sem1,
              tbl, acc)
    plsc.subcore_barrier()
    _agg_pass(tb_hbm, outb_hbm, c, s, srcv, dstv, rows0, rows1, sem0, sem1,
              tbl, acc)


_agg64x2 = pl.kernel(
    _agg2_body,
    out_type=[jax.ShapeDtypeStruct((NC * N_PAD, AGG_D), jnp.float32),
              jax.ShapeDtypeStruct((NC * N_PAD, AGG_D), jnp.float32)],
    mesh=_mesh,
    scratch_types=list(_AGG_SCRATCH),
    compiler_params=_sc_params,
)


_BLK = 512
_GRID = (N_NODES + _BLK - 1) // _BLK


def _tc1_body(x_ref, w_ref, da_ref, db_ref, t1a_ref, t1b_ref):
    dinv = lax.rsqrt(da_ref[:, :1] + db_ref[:, :1] + 1.0)
    xw = jnp.dot(x_ref[...], w_ref[...], preferred_element_type=jnp.float32)
    t1 = xw * dinv
    t1a_ref[...] = t1[:, :AGG_D]
    t1b_ref[...] = t1[:, AGG_D:]


_tc1 = pl.pallas_call(
    _tc1_body,
    grid=(_GRID,),
    in_specs=[
        pl.BlockSpec((_BLK, IN_DIM), lambda i: (i, 0)),
        pl.BlockSpec((IN_DIM, HID_DIM), lambda i: (0, 0)),
        pl.BlockSpec((_BLK, 16), lambda i: (i, 0)),
        pl.BlockSpec((_BLK, 16), lambda i: (i, 0)),
    ],
    out_specs=[
        pl.BlockSpec((_BLK, AGG_D), lambda i: (i, 0)),
        pl.BlockSpec((_BLK, AGG_D), lambda i: (i, 0)),
    ],
    out_shape=[
        jax.ShapeDtypeStruct((N_PAD, AGG_D), jnp.float32),
        jax.ShapeDtypeStruct((N_PAD, AGG_D), jnp.float32),
    ],
)


def _tc2_body(aaa_ref, aba_ref, aab_ref, abb_ref, t1a_ref, t1b_ref,
              da_ref, db_ref, b1_ref, w2_ref, t2_ref):
    dinv = lax.rsqrt(da_ref[:, :1] + db_ref[:, :1] + 1.0)
    b1 = b1_ref[...]
    ha = (aaa_ref[...] + aba_ref[...] + t1a_ref[...]) * dinv + b1[:, :AGG_D]
    hb = (aab_ref[...] + abb_ref[...] + t1b_ref[...]) * dinv + b1[:, AGG_D:]
    ha = jnp.maximum(ha, 0.0)
    hb = jnp.maximum(hb, 0.0)
    w2 = w2_ref[...]
    t2 = (jnp.dot(ha, w2[:AGG_D, :], preferred_element_type=jnp.float32) +
          jnp.dot(hb, w2[AGG_D:, :], preferred_element_type=jnp.float32))
    t2_ref[...] = t2 * dinv


_tc2 = pl.pallas_call(
    _tc2_body,
    grid=(_GRID,),
    in_specs=[
        pl.BlockSpec((_BLK, AGG_D), lambda i: (i, 0)),
        pl.BlockSpec((_BLK, AGG_D), lambda i: (i, 0)),
        pl.BlockSpec((_BLK, AGG_D), lambda i: (i, 0)),
        pl.BlockSpec((_BLK, AGG_D), lambda i: (i, 0)),
        pl.BlockSpec((_BLK, AGG_D), lambda i: (i, 0)),
        pl.BlockSpec((_BLK, AGG_D), lambda i: (i, 0)),
        pl.BlockSpec((_BLK, 16), lambda i: (i, 0)),
        pl.BlockSpec((_BLK, 16), lambda i: (i, 0)),
        pl.BlockSpec((1, HID_DIM), lambda i: (0, 0)),
        pl.BlockSpec((HID_DIM, OUT_DIM), lambda i: (0, 0)),
    ],
    out_specs=pl.BlockSpec((_BLK, OUT_DIM), lambda i: (i, 0)),
    out_shape=jax.ShapeDtypeStruct((N_PAD, OUT_DIM), jnp.float32),
)


def _tc3_body(aa_ref, ab_ref, t2_ref, da_ref, db_ref, b2_ref, out_ref):
    dinv = lax.rsqrt(da_ref[:, :1] + db_ref[:, :1] + 1.0)
    out_ref[...] = (aa_ref[...] + ab_ref[...] + t2_ref[...]) * dinv + b2_ref[...]


_tc3 = pl.pallas_call(
    _tc3_body,
    grid=(_GRID,),
    in_specs=[
        pl.BlockSpec((_BLK, OUT_DIM), lambda i: (i, 0)),
        pl.BlockSpec((_BLK, OUT_DIM), lambda i: (i, 0)),
        pl.BlockSpec((_BLK, OUT_DIM), lambda i: (i, 0)),
        pl.BlockSpec((_BLK, 16), lambda i: (i, 0)),
        pl.BlockSpec((_BLK, 16), lambda i: (i, 0)),
        pl.BlockSpec((1, OUT_DIM), lambda i: (0, 0)),
    ],
    out_specs=pl.BlockSpec((_BLK, OUT_DIM), lambda i: (i, 0)),
    out_shape=jax.ShapeDtypeStruct((N_NODES, OUT_DIM), jnp.float32),
)


def kernel(x, edge_index, W1, b1, W2, b2):
    src = edge_index[0].astype(jnp.int32)
    dst = edge_index[1].astype(jnp.int32)
    pad = E_PAD - N_EDGES
    src_r = jnp.concatenate([src, jnp.zeros((pad,), jnp.int32)])
    dst_r = jnp.concatenate([dst, jnp.full((pad,), DUMMY_ROW, jnp.int32)])
    src_c = src_r.reshape(NW, EDGES_PER_TILE // CHUNK, CHUNK)
    dst_c = dst_r.reshape(NW, EDGES_PER_TILE // CHUNK, CHUNK)

    deg2 = _deg_call(dst_c)
    da = deg2[:N_NODES, :]
    db = deg2[N_PAD:N_PAD + N_NODES, :]

    t1a, t1b = _tc1(x, W1, da, db)
    agg1a, agg1b = _agg64x2(t1a, t1b, src_c, dst_c)
    t2 = _tc2(agg1a[:N_NODES], agg1a[N_PAD:N_PAD + N_NODES],
              agg1b[:N_NODES], agg1b[N_PAD:N_PAD + N_NODES],
              t1a[:N_NODES], t1b[:N_NODES], da, db,
              b1.reshape(1, HID_DIM), W2)
    agg2 = _agg64(t2, src_c, dst_c)
    out = _tc3(agg2[:N_NODES], agg2[N_PAD:N_PAD + N_NODES], t2[:N_NODES], da, db,
               b2.reshape(1, OUT_DIM))
    return out


# speedup vs baseline: 1.1331x; 1.1331x over previous
import functools

import jax
import jax.numpy as jnp
from jax import lax
from jax.experimental import pallas as pl
from jax.experimental.pallas import tpu as pltpu
from jax.experimental.pallas import tpu_sc as plsc

N_NODES = 10000
IN_DIM = 128
HID_DIM = 128
OUT_DIM = 64
N_EDGES = 320000

NC = 2
NS = 16
NW = NC * NS
CHUNK = 128
CHUNKS_PER_TILE = 80
EDGES_PER_TILE = CHUNK * CHUNKS_PER_TILE
E_PAD = NW * EDGES_PER_TILE
N_PAD = 10240
ROWS_PER_TILE = N_PAD // NS
DUMMY_ROW = N_NODES

_mesh = plsc.VectorSubcoreMesh(core_axis_name="c", subcore_axis_name="s")
_sc_params = pltpu.CompilerParams(use_tc_tiling_on_sc=False)


def _deg_body(dst_hbm, deg_hbm, dstv, ones_v, stage_v, sem, acc):
    c = lax.axis_index("c")
    s = lax.axis_index("s")
    wid = c * NS + s
    pltpu.sync_copy(dst_hbm.at[wid], dstv)

    zero16 = jnp.zeros((16,), jnp.float32)
    one16 = jnp.ones((16,), jnp.float32)

    def fill_ones(r, _):
        ones_v[r, :] = one16
        return 0

    lax.fori_loop(0, CHUNK, fill_ones, 0)

    def fill_zero(r, _):
        stage_v[r, :] = zero16
        return 0

    lax.fori_loop(0, ROWS_PER_TILE, fill_zero, 0)
    pltpu.sync_copy(stage_v, acc.at[pl.ds(s * ROWS_PER_TILE, ROWS_PER_TILE)])
    plsc.subcore_barrier()

    def step(j, _):
        pltpu.async_copy(ones_v, acc.at[dstv.at[j]], sem, add=True)
        return 0

    lax.fori_loop(0, CHUNKS_PER_TILE, step, 0)

    def drain(j, _):
        pltpu.make_async_copy(ones_v, acc.at[dstv.at[0]], sem).wait()
        return 0

    lax.fori_loop(0, CHUNKS_PER_TILE, drain, 0)
    plsc.subcore_barrier()

    off = s * ROWS_PER_TILE
    pltpu.sync_copy(acc.at[pl.ds(off, ROWS_PER_TILE)],
                    deg_hbm.at[pl.ds(c * N_PAD + off, ROWS_PER_TILE)])


_deg_call = pl.kernel(
    _deg_body,
    out_type=jax.ShapeDtypeStruct((NC * N_PAD, 16), jnp.float32),
    mesh=_mesh,
    scratch_types=[
        pltpu.VMEM((CHUNKS_PER_TILE, CHUNK), jnp.int32),
        pltpu.VMEM((CHUNK, 16), jnp.float32),
        pltpu.VMEM((ROWS_PER_TILE, 16), jnp.float32),
        pltpu.SemaphoreType.DMA,
        pltpu.VMEM_SHARED((N_PAD, 16), jnp.float32),
    ],
    compiler_params=_sc_params,
)


AGG_D = 64


def _agg_pass(table_hbm, out_hbm, c, s, srcv, dstv, rows0, rows1, sem0, sem1,
              tbl, acc):
    ch = CHUNK
    npc = EDGES_PER_TILE // ch
    zero16 = jnp.zeros((16,), jnp.float32)

    def fill_zero(r, _):
        for k in range(AGG_D // 16):
            rows0[r, pl.ds(k * 16, 16)] = zero16
        return 0

    lax.fori_loop(0, ch, fill_zero, 0)
    base = s * ROWS_PER_TILE
    for k in range(ROWS_PER_TILE // ch):
        pltpu.sync_copy(rows0, acc.at[pl.ds(base + k * ch, ch)])
    pltpu.sync_copy(table_hbm.at[pl.ds(base, ROWS_PER_TILE)],
                    tbl.at[pl.ds(base, ROWS_PER_TILE)])
    plsc.subcore_barrier()

    pltpu.async_copy(tbl.at[srcv.at[0]], rows0, sem0)
    n2 = npc // 2

    def step(i, _):
        j0 = 2 * i
        pltpu.async_copy(tbl.at[srcv.at[j0 + 1]], rows1, sem1)
        pltpu.make_async_copy(tbl.at[srcv.at[j0]], rows0, sem0).wait()
        pltpu.sync_copy(rows0, acc.at[dstv.at[j0]], add=True)
        pltpu.async_copy(tbl.at[srcv.at[j0 + 2]], rows0, sem0)
        pltpu.make_async_copy(tbl.at[srcv.at[j0 + 1]], rows1, sem1).wait()
        pltpu.sync_copy(rows1, acc.at[dstv.at[j0 + 1]], add=True)
        return 0

    lax.fori_loop(0, n2, step, 0)
    pltpu.make_async_copy(tbl.at[srcv.at[npc]], rows0, sem0).wait()
    plsc.subcore_barrier()

    pltpu.sync_copy(acc.at[pl.ds(base, ROWS_PER_TILE)],
                    out_hbm.at[pl.ds(c * N_PAD + base, ROWS_PER_TILE)])


def _load_indices(src_hbm, dst_hbm, wid, srcv, dstv):
    npc = EDGES_PER_TILE // CHUNK
    pltpu.sync_copy(src_hbm.at[wid], srcv.at[pl.ds(0, npc)])
    pltpu.sync_copy(dst_hbm.at[wid], dstv)
    zero16i = jnp.zeros((16,), jnp.int32)
    for k in range(CHUNK // 16):
        srcv[npc, pl.ds(k * 16, 16)] = zero16i


_AGG_SCRATCH = [
    pltpu.VMEM((EDGES_PER_TILE // CHUNK + 1, CHUNK), jnp.int32),
    pltpu.VMEM((EDGES_PER_TILE // CHUNK, CHUNK), jnp.int32),
    pltpu.VMEM((CHUNK, AGG_D), jnp.float32),
    pltpu.VMEM((CHUNK, AGG_D), jnp.float32),
    pltpu.SemaphoreType.DMA,
    pltpu.SemaphoreType.DMA,
    pltpu.VMEM_SHARED((N_PAD, AGG_D), jnp.float32),
    pltpu.VMEM_SHARED((N_PAD, AGG_D), jnp.float32),
]


def _agg_body(table_hbm, src_hbm, dst_hbm, out_hbm, srcv, dstv, rows0, rows1,
              sem0, sem1, tbl, acc):
    c = lax.axis_index("c")
    s = lax.axis_index("s")
    _load_indices(src_hbm, dst_hbm, c * NS + s, srcv, dstv)
    _agg_pass(table_hbm, out_hbm, c, s, srcv, dstv, rows0, rows1, sem0, sem1,
              tbl, acc)


_agg64 = pl.kernel(
    _agg_body,
    out_type=jax.ShapeDtypeStruct((NC * N_PAD, AGG_D), jnp.float32),
    mesh=_mesh,
    scratch_types=list(_AGG_SCRATCH),
    compiler_params=_sc_params,
)


def _agg2_body(ta_hbm, tb_hbm, src_hbm, dst_hbm, outa_hbm, outb_hbm,
               srcv, dstv, rows0, rows1, sem0, sem1, tbl, acc):
    c = lax.axis_index("c")
    s = lax.axis_index("s")
    _load_indices(src_hbm, dst_hbm, c * NS + s, srcv, dstv)
    _agg_pass(ta_hbm, outa_hbm, c, s, srcv, dstv, rows0, rows1, sem0, sem1,
              tbl, acc)
    plsc.subcore_barrier()
    _agg_pass(tb_hbm, outb_hbm, c, s, srcv, dstv, rows0, rows1, sem0, sem1,
              tbl, acc)


_agg64x2 = pl.kernel(
    _agg2_body,
    out_type=[jax.ShapeDtypeStruct((NC * N_PAD, AGG_D), jnp.float32),
              jax.ShapeDtypeStruct((NC * N_PAD, AGG_D), jnp.float32)],
    mesh=_mesh,
    scratch_types=list(_AGG_SCRATCH),
    compiler_params=_sc_params,
)


_BLK = 2048
_GRID = N_PAD // _BLK
_HB = N_PAD // _BLK


def _tc1_body(x_ref, w_ref, da_ref, db_ref, t1a_ref, t1b_ref):
    dinv = lax.rsqrt(da_ref[:, :1] + db_ref[:, :1] + 1.0)
    xw = jnp.dot(x_ref[...], w_ref[...], preferred_element_type=jnp.float32)
    t1 = xw * dinv
    t1a_ref[...] = t1[:, :AGG_D]
    t1b_ref[...] = t1[:, AGG_D:]


_tc1 = pl.pallas_call(
    _tc1_body,
    grid=(_GRID,),
    in_specs=[
        pl.BlockSpec((_BLK, IN_DIM), lambda i: (i, 0)),
        pl.BlockSpec((IN_DIM, HID_DIM), lambda i: (0, 0)),
        pl.BlockSpec((_BLK, 16), lambda i: (i, 0)),
        pl.BlockSpec((_BLK, 16), lambda i: (i + _HB, 0)),
    ],
    out_specs=[
        pl.BlockSpec((_BLK, AGG_D), lambda i: (i, 0)),
        pl.BlockSpec((_BLK, AGG_D), lambda i: (i, 0)),
    ],
    out_shape=[
        jax.ShapeDtypeStruct((N_PAD, AGG_D), jnp.float32),
        jax.ShapeDtypeStruct((N_PAD, AGG_D), jnp.float32),
    ],
)


def _tc2_body(aaa_ref, aba_ref, aab_ref, abb_ref, t1a_ref, t1b_ref,
              da_ref, db_ref, b1_ref, w2_ref, t2_ref):
    dinv = lax.rsqrt(da_ref[:, :1] + db_ref[:, :1] + 1.0)
    b1 = b1_ref[...]
    ha = (aaa_ref[...] + aba_ref[...] + t1a_ref[...]) * dinv + b1[:, :AGG_D]
    hb = (aab_ref[...] + abb_ref[...] + t1b_ref[...]) * dinv + b1[:, AGG_D:]
    ha = jnp.maximum(ha, 0.0)
    hb = jnp.maximum(hb, 0.0)
    w2 = w2_ref[...]
    t2 = (jnp.dot(ha, w2[:AGG_D, :], preferred_element_type=jnp.float32) +
          jnp.dot(hb, w2[AGG_D:, :], preferred_element_type=jnp.float32))
    t2_ref[...] = t2 * dinv


_tc2 = pl.pallas_call(
    _tc2_body,
    grid=(_GRID,),
    in_specs=[
        pl.BlockSpec((_BLK, AGG_D), lambda i: (i, 0)),
        pl.BlockSpec((_BLK, AGG_D), lambda i: (i + _HB, 0)),
        pl.BlockSpec((_BLK, AGG_D), lambda i: (i, 0)),
        pl.BlockSpec((_BLK, AGG_D), lambda i: (i + _HB, 0)),
        pl.BlockSpec((_BLK, AGG_D), lambda i: (i, 0)),
        pl.BlockSpec((_BLK, AGG_D), lambda i: (i, 0)),
        pl.BlockSpec((_BLK, 16), lambda i: (i, 0)),
        pl.BlockSpec((_BLK, 16), lambda i: (i + _HB, 0)),
        pl.BlockSpec((1, HID_DIM), lambda i: (0, 0)),
        pl.BlockSpec((HID_DIM, OUT_DIM), lambda i: (0, 0)),
    ],
    out_specs=pl.BlockSpec((_BLK, OUT_DIM), lambda i: (i, 0)),
    out_shape=jax.ShapeDtypeStruct((N_PAD, OUT_DIM), jnp.float32),
)


def _tc3_body(aa_ref, ab_ref, t2_ref, da_ref, db_ref, b2_ref, out_ref):
    dinv = lax.rsqrt(da_ref[:, :1] + db_ref[:, :1] + 1.0)
    out_ref[...] = (aa_ref[...] + ab_ref[...] + t2_ref[...]) * dinv + b2_ref[...]


_tc3 = pl.pallas_call(
    _tc3_body,
    grid=(_GRID,),
    in_specs=[
        pl.BlockSpec((_BLK, OUT_DIM), lambda i: (i, 0)),
        pl.BlockSpec((_BLK, OUT_DIM), lambda i: (i + _HB, 0)),
        pl.BlockSpec((_BLK, OUT_DIM), lambda i: (i, 0)),
        pl.BlockSpec((_BLK, 16), lambda i: (i, 0)),
        pl.BlockSpec((_BLK, 16), lambda i: (i + _HB, 0)),
        pl.BlockSpec((1, OUT_DIM), lambda i: (0, 0)),
    ],
    out_specs=pl.BlockSpec((_BLK, OUT_DIM), lambda i: (i, 0)),
    out_shape=jax.ShapeDtypeStruct((N_NODES, OUT_DIM), jnp.float32),
)


def kernel(x, edge_index, W1, b1, W2, b2):
    src = edge_index[0].astype(jnp.int32)
    dst = edge_index[1].astype(jnp.int32)
    pad = E_PAD - N_EDGES
    src_r = jnp.concatenate([src, jnp.zeros((pad,), jnp.int32)])
    dst_r = jnp.concatenate([dst, jnp.full((pad,), DUMMY_ROW, jnp.int32)])
    src_c = src_r.reshape(NW, EDGES_PER_TILE // CHUNK, CHUNK)
    dst_c = dst_r.reshape(NW, EDGES_PER_TILE // CHUNK, CHUNK)

    deg2 = _deg_call(dst_c)

    t1a, t1b = _tc1(x, W1, deg2, deg2)
    agg1a, agg1b = _agg64x2(t1a, t1b, src_c, dst_c)
    t2 = _tc2(agg1a, agg1a, agg1b, agg1b, t1a, t1b, deg2, deg2,
              b1.reshape(1, HID_DIM), W2)
    agg2 = _agg64(t2, src_c, dst_c)
    out = _tc3(agg2, agg2, t2, deg2, deg2,
               b2.reshape(1, OUT_DIM))
    return out


# speedup vs baseline: 1.3258x; 1.1701x over previous
import functools

import jax
import jax.numpy as jnp
from jax import lax
from jax.experimental import pallas as pl
from jax.experimental.pallas import tpu as pltpu
from jax.experimental.pallas import tpu_sc as plsc

N_NODES = 10000
IN_DIM = 128
HID_DIM = 128
OUT_DIM = 64
N_EDGES = 320000

NC = 2
NS = 16
NW = NC * NS
CHUNK = 80
EDGES_PER_TILE = N_EDGES // NW
CHUNKS_PER_TILE = EDGES_PER_TILE // CHUNK
N_PAD = 10240
ROWS_PER_TILE = N_PAD // NS

_mesh = plsc.VectorSubcoreMesh(core_axis_name="c", subcore_axis_name="s")
_sc_params = pltpu.CompilerParams(use_tc_tiling_on_sc=False)


def _deg_body(dst_hbm, deg_hbm, dstv, ones_v, stage_v, sem, acc):
    c = lax.axis_index("c")
    s = lax.axis_index("s")
    wid = c * NS + s
    pltpu.sync_copy(dst_hbm.at[wid], dstv)

    zero16 = jnp.zeros((16,), jnp.float32)
    one16 = jnp.ones((16,), jnp.float32)

    def fill_ones(r, _):
        ones_v[r, :] = one16
        return 0

    lax.fori_loop(0, CHUNK, fill_ones, 0)

    def fill_zero(r, _):
        stage_v[r, :] = zero16
        return 0

    lax.fori_loop(0, ROWS_PER_TILE, fill_zero, 0)
    pltpu.sync_copy(stage_v, acc.at[pl.ds(s * ROWS_PER_TILE, ROWS_PER_TILE)])
    plsc.subcore_barrier()

    def step(j, _):
        pltpu.async_copy(ones_v, acc.at[dstv.at[j]], sem, add=True)
        return 0

    lax.fori_loop(0, CHUNKS_PER_TILE, step, 0)

    def drain(j, _):
        pltpu.make_async_copy(ones_v, acc.at[dstv.at[0]], sem).wait()
        return 0

    lax.fori_loop(0, CHUNKS_PER_TILE, drain, 0)
    plsc.subcore_barrier()

    off = s * ROWS_PER_TILE
    pltpu.sync_copy(acc.at[pl.ds(off, ROWS_PER_TILE)],
                    deg_hbm.at[pl.ds(c * N_PAD + off, ROWS_PER_TILE)])


_deg_call = pl.kernel(
    _deg_body,
    out_type=jax.ShapeDtypeStruct((NC * N_PAD, 16), jnp.float32),
    mesh=_mesh,
    scratch_types=[
        pltpu.VMEM((CHUNKS_PER_TILE, CHUNK), jnp.int32),
        pltpu.VMEM((CHUNK, 16), jnp.float32),
        pltpu.VMEM((ROWS_PER_TILE, 16), jnp.float32),
        pltpu.SemaphoreType.DMA,
        pltpu.VMEM_SHARED((N_PAD, 16), jnp.float32),
    ],
    compiler_params=_sc_params,
)


AGG_D = 64


def _agg_pass(table_hbm, tcol, out_hbm, orow, ocol, c, s, srcv, dstv,
              rows0, rows1, sem0, sem1, tbl, acc):
    ch = CHUNK
    npc = EDGES_PER_TILE // ch
    zero16 = jnp.zeros((16,), jnp.float32)

    def fill_zero(r, _):
        for k in range(AGG_D // 16):
            rows0[r, pl.ds(k * 16, 16)] = zero16
        return 0

    lax.fori_loop(0, ch, fill_zero, 0)
    base = s * ROWS_PER_TILE
    for k in range(ROWS_PER_TILE // ch):
        pltpu.sync_copy(rows0, acc.at[pl.ds(base + k * ch, ch)])
    pltpu.sync_copy(table_hbm.at[pl.ds(base, ROWS_PER_TILE),
                                 pl.ds(tcol * AGG_D, AGG_D)],
                    tbl.at[pl.ds(base, ROWS_PER_TILE)])
    plsc.subcore_barrier()

    pltpu.async_copy(tbl.at[srcv.at[0]], rows0, sem0)
    n2 = npc // 2

    def step(i, _):
        j0 = 2 * i
        pltpu.async_copy(tbl.at[srcv.at[j0 + 1]], rows1, sem1)
        pltpu.make_async_copy(tbl.at[srcv.at[j0]], rows0, sem0).wait()
        pltpu.sync_copy(rows0, acc.at[dstv.at[j0]], add=True)
        pltpu.async_copy(tbl.at[srcv.at[j0 + 2]], rows0, sem0)
        pltpu.make_async_copy(tbl.at[srcv.at[j0 + 1]], rows1, sem1).wait()
        pltpu.sync_copy(rows1, acc.at[dstv.at[j0 + 1]], add=True)
        return 0

    lax.fori_loop(0, n2, step, 0)
    pltpu.make_async_copy(tbl.at[srcv.at[npc - 1]], rows0, sem0).wait()
    pltpu.sync_copy(rows0, acc.at[dstv.at[npc - 1]], add=True)
    plsc.subcore_barrier()

    pltpu.sync_copy(acc.at[pl.ds(base, ROWS_PER_TILE)],
                    out_hbm.at[pl.ds(orow + base, ROWS_PER_TILE),
                               pl.ds(ocol * AGG_D, AGG_D)])


def _load_indices(src_hbm, dst_hbm, wid, srcv, dstv):
    pltpu.sync_copy(src_hbm.at[wid], srcv)
    pltpu.sync_copy(dst_hbm.at[wid], dstv)


_AGG_SCRATCH = [
    pltpu.VMEM((CHUNKS_PER_TILE, CHUNK), jnp.int32),
    pltpu.VMEM((CHUNKS_PER_TILE, CHUNK), jnp.int32),
    pltpu.VMEM((CHUNK, AGG_D), jnp.float32),
    pltpu.VMEM((CHUNK, AGG_D), jnp.float32),
    pltpu.SemaphoreType.DMA,
    pltpu.SemaphoreType.DMA,
    pltpu.VMEM_SHARED((N_PAD, AGG_D), jnp.float32),
    pltpu.VMEM_SHARED((N_PAD, AGG_D), jnp.float32),
]


def _agg_body(table_hbm, src_hbm, dst_hbm, out_hbm, srcv, dstv, rows0, rows1,
              sem0, sem1, tbl, acc):
    c = lax.axis_index("c")
    s = lax.axis_index("s")
    _load_indices(src_hbm, dst_hbm, c * NS + s, srcv, dstv)
    _agg_pass(table_hbm, 0, out_hbm, 0, c, c, s, srcv, dstv, rows0, rows1,
              sem0, sem1, tbl, acc)


_agg64 = pl.kernel(
    _agg_body,
    out_type=jax.ShapeDtypeStruct((N_PAD, HID_DIM), jnp.float32),
    mesh=_mesh,
    scratch_types=list(_AGG_SCRATCH),
    compiler_params=_sc_params,
)


def _agg2_body(t1_hbm, src_hbm, dst_hbm, out_hbm,
               srcv, dstv, rows0, rows1, sem0, sem1, tbl, acc):
    c = lax.axis_index("c")
    s = lax.axis_index("s")
    _load_indices(src_hbm, dst_hbm, c * NS + s, srcv, dstv)
    _agg_pass(t1_hbm, 0, out_hbm, c * N_PAD, 0, c, s, srcv, dstv,
              rows0, rows1, sem0, sem1, tbl, acc)
    plsc.subcore_barrier()
    _agg_pass(t1_hbm, 1, out_hbm, c * N_PAD, 1, c, s, srcv, dstv,
              rows0, rows1, sem0, sem1, tbl, acc)


_agg64x2 = pl.kernel(
    _agg2_body,
    out_type=jax.ShapeDtypeStruct((NC * N_PAD, HID_DIM), jnp.float32),
    mesh=_mesh,
    scratch_types=list(_AGG_SCRATCH),
    compiler_params=_sc_params,
)


_BLK = 2048
_GRID = N_PAD // _BLK
_HB = N_PAD // _BLK


def _tc1_body(x_ref, w_ref, da_ref, db_ref, t1_ref):
    dinv = lax.rsqrt(da_ref[:, :1] + db_ref[:, :1] + 1.0)
    xw = jnp.dot(x_ref[...], w_ref[...], preferred_element_type=jnp.float32)
    t1_ref[...] = xw * dinv


_tc1 = pl.pallas_call(
    _tc1_body,
    grid=(_GRID,),
    in_specs=[
        pl.BlockSpec((_BLK, IN_DIM), lambda i: (i, 0)),
        pl.BlockSpec((IN_DIM, HID_DIM), lambda i: (0, 0)),
        pl.BlockSpec((_BLK, 16), lambda i: (i, 0)),
        pl.BlockSpec((_BLK, 16), lambda i: (i + _HB, 0)),
    ],
    out_specs=pl.BlockSpec((_BLK, HID_DIM), lambda i: (i, 0)),
    out_shape=jax.ShapeDtypeStruct((N_PAD, HID_DIM), jnp.float32),
)


def _tc2_body(agg0_ref, agg1_ref, t1_ref, da_ref, db_ref, b1_ref, w2_ref,
              t2_ref):
    dinv = lax.rsqrt(da_ref[:, :1] + db_ref[:, :1] + 1.0)
    h = (agg0_ref[...] + agg1_ref[...] + t1_ref[...]) * dinv + b1_ref[...]
    h = jnp.maximum(h, 0.0)
    t2_ref[...] = jnp.dot(h, w2_ref[...], preferred_element_type=jnp.float32) * dinv


_tc2 = pl.pallas_call(
    _tc2_body,
    grid=(_GRID,),
    in_specs=[
        pl.BlockSpec((_BLK, HID_DIM), lambda i: (i, 0)),
        pl.BlockSpec((_BLK, HID_DIM), lambda i: (i + _HB, 0)),
        pl.BlockSpec((_BLK, HID_DIM), lambda i: (i, 0)),
        pl.BlockSpec((_BLK, 16), lambda i: (i, 0)),
        pl.BlockSpec((_BLK, 16), lambda i: (i + _HB, 0)),
        pl.BlockSpec((1, HID_DIM), lambda i: (0, 0)),
        pl.BlockSpec((HID_DIM, HID_DIM), lambda i: (0, 0)),
    ],
    out_specs=pl.BlockSpec((_BLK, HID_DIM), lambda i: (i, 0)),
    out_shape=jax.ShapeDtypeStruct((N_PAD, HID_DIM), jnp.float32),
)


def _tc3_body(agg_ref, t2_ref, da_ref, db_ref, b2_ref, out_ref):
    dinv = lax.rsqrt(da_ref[:, :1] + db_ref[:, :1] + 1.0)
    agg = agg_ref[...]
    out_ref[...] = ((agg[:, :OUT_DIM] + agg[:, OUT_DIM:] + t2_ref[:, :OUT_DIM])
                    * dinv + b2_ref[...])


_tc3 = pl.pallas_call(
    _tc3_body,
    grid=(_GRID,),
    in_specs=[
        pl.BlockSpec((_BLK, HID_DIM), lambda i: (i, 0)),
        pl.BlockSpec((_BLK, HID_DIM), lambda i: (i, 0)),
        pl.BlockSpec((_BLK, 16), lambda i: (i, 0)),
        pl.BlockSpec((_BLK, 16), lambda i: (i + _HB, 0)),
        pl.BlockSpec((1, OUT_DIM), lambda i: (0, 0)),
    ],
    out_specs=pl.BlockSpec((_BLK, OUT_DIM), lambda i: (i, 0)),
    out_shape=jax.ShapeDtypeStruct((N_NODES, OUT_DIM), jnp.float32),
)


def kernel(x, edge_index, W1, b1, W2, b2):
    ei = edge_index.astype(jnp.int32)
    src_c = ei[0].reshape(NW, CHUNKS_PER_TILE, CHUNK)
    dst_c = ei[1].reshape(NW, CHUNKS_PER_TILE, CHUNK)

    deg2 = _deg_call(dst_c)

    t1 = _tc1(x, W1, deg2, deg2)
    agg1p = _agg64x2(t1, src_c, dst_c)
    w2cat = jnp.concatenate([W2, W2], axis=1)
    t2x = _tc2(agg1p, agg1p, t1, deg2, deg2,
               b1.reshape(1, HID_DIM), w2cat)
    agg2p = _agg64(t2x, src_c, dst_c)
    out = _tc3(agg2p, t2x, deg2, deg2, b2.reshape(1, OUT_DIM))
    return out
